# baseline (speedup 1.0000x reference)
import functools

import jax
import jax.numpy as jnp
from jax import lax
from jax.experimental import pallas as pl
from jax.experimental.pallas import tpu as pltpu
from jax.experimental.pallas import tpu_sc as plsc

AFL = 32
NBL = 4
N = 50000
M = 16
N0 = 1000
A = 50
EPS = 1e-5

NW = 32
CHUNK = 128
KCH = 196
PER_W = KCH * CHUNK
E_PAD = NW * PER_W
E = N * M

BA = 1000
BE = BA * M
GRID = N // BA


def _lrelu(x):
    return jnp.maximum(x, 0.01 * x)


def _sc_gather(x, idx3d):
    mesh = plsc.VectorSubcoreMesh(core_axis_name="c", subcore_axis_name="s")

    @functools.partial(
        pl.kernel,
        mesh=mesh,
        out_type=jax.ShapeDtypeStruct((E_PAD, AFL), jnp.float32),
        scratch_types=[
            pltpu.VMEM((KCH, CHUNK), jnp.int32),
            pltpu.VMEM((CHUNK, AFL), jnp.float32),
            pltpu.VMEM((CHUNK, AFL), jnp.float32),
            pltpu.SemaphoreType.DMA,
            pltpu.SemaphoreType.DMA,
        ],
        compiler_params=pltpu.CompilerParams(use_tc_tiling_on_sc=False),
    )
    def k(x_hbm, idx_hbm, g_hbm, idx_v, rows0, rows1, sem0, sem1):
        wid = lax.axis_index("s") * 2 + lax.axis_index("c")
        pltpu.sync_copy(idx_hbm.at[wid], idx_v)
        base = wid * PER_W

        def issue(j, rows, sem):
            return pltpu.async_copy(x_hbm.at[idx_v.at[j]], rows, sem)

        def drain(j, rows):
            pltpu.sync_copy(rows, g_hbm.at[pl.ds(base + j * CHUNK, CHUNK)])

        c0 = issue(0, rows0, sem0)

        def body(jj, carry):
            j = jj * 2

            c_next = issue(j + 1, rows1, sem1)
            c0 = pltpu.make_async_copy(x_hbm.at[idx_v.at[j]], rows0, sem0)
            c0.wait()
            drain(j, rows0)

            c_next2 = issue(j + 2, rows0, sem0)
            c1 = pltpu.make_async_copy(x_hbm.at[idx_v.at[j + 1]], rows1, sem1)
            c1.wait()
            drain(j + 1, rows1)
            return carry

        lax.fori_loop(0, (KCH - 2) // 2, body, 0, unroll=False)
        j = KCH - 2
        c_last = issue(j + 1, rows1, sem1)
        pltpu.make_async_copy(x_hbm.at[idx_v.at[j]], rows0, sem0).wait()
        drain(j, rows0)
        pltpu.make_async_copy(x_hbm.at[idx_v.at[j + 1]], rows1, sem1).wait()
        drain(j + 1, rows1)

    return k(x, idx3d)


def _embed(atom_fea, WeT, be):
    def body(a_ref, w_ref, b_ref, o_ref):
        o_ref[...] = jnp.dot(a_ref[...], w_ref[...],
                             preferred_element_type=jnp.float32) + b_ref[...]

    return pl.pallas_call(
        body,
        grid=(GRID,),
        in_specs=[
            pl.BlockSpec((BA, 128), lambda i: (i, 0)),
            pl.BlockSpec((128, AFL), lambda i: (0, 0)),
            pl.BlockSpec((1, AFL), lambda i: (0, 0)),
        ],
        out_specs=pl.BlockSpec((BA, AFL), lambda i: (i, 0)),
        out_shape=jax.ShapeDtypeStruct((N, AFL), jnp.float32),
    )(atom_fea, WeT, be)


def _gate_preact(x_blk, g_blk, nb_blk, wa, wb, wc, bf):
    self_g = jnp.dot(x_blk, wa, preferred_element_type=jnp.float32)
    nb_term = lax.dot_general(nb_blk, wc, (((0,), (0,)), ((), ())),
                              preferred_element_type=jnp.float32)
    edge = (jnp.dot(g_blk, wb, preferred_element_type=jnp.float32)
            + nb_term + bf)
    self_rep = jnp.broadcast_to(self_g[:, None, :], (BA, M, 2 * AFL))
    self_rep = self_rep.reshape(BE, 2 * AFL)
    return self_rep + edge


def _stats1(x, g, nbrf2, WaT, WbT, WcT, bf):
    def body(x_ref, g_ref, nb_ref, wa_ref, wb_ref, wc_ref, bf_ref, st_ref):
        gated = _gate_preact(x_ref[...], g_ref[...], nb_ref[...],
                             wa_ref[...], wb_ref[...], wc_ref[...], bf_ref[...])

        @pl.when(pl.program_id(0) == 0)
        def _():
            st_ref[...] = jnp.zeros_like(st_ref)

        s1 = jnp.sum(gated, axis=0, keepdims=True)
        s2 = jnp.sum(gated * gated, axis=0, keepdims=True)
        st_ref[...] += jnp.concatenate([s1, s2], axis=0)

    return pl.pallas_call(
        body,
        grid=(GRID,),
        in_specs=[
            pl.BlockSpec((BA, AFL), lambda i: (i, 0)),
            pl.BlockSpec((BE, AFL), lambda i: (i, 0)),
            pl.BlockSpec((NBL, BE), lambda i: (0, i)),
            pl.BlockSpec((AFL, 2 * AFL), lambda i: (0, 0)),
            pl.BlockSpec((AFL, 2 * AFL), lambda i: (0, 0)),
            pl.BlockSpec((NBL, 2 * AFL), lambda i: (0, 0)),
            pl.BlockSpec((1, 2 * AFL), lambda i: (0, 0)),
        ],
        out_specs=pl.BlockSpec((2, 2 * AFL), lambda i: (0, 0)),
        out_shape=jax.ShapeDtypeStruct((2, 2 * AFL), jnp.float32),
        compiler_params=pltpu.CompilerParams(
            dimension_semantics=("arbitrary",)),
    )(x, g, nbrf2, WaT, WbT, WcT, bf)


def _conv_sum(x, g, nbrf2, WaT, WbT, WcT, bf, st1, g1, b1):
    def body(x_ref, g_ref, nb_ref, wa_ref, wb_ref, wc_ref, bf_ref,
             st_ref, g1_ref, b1_ref, s_ref, st2_ref):
        mean = st_ref[0:1, :] / E
        var = st_ref[1:2, :] / E - mean * mean
        sc = lax.rsqrt(var + EPS) * g1_ref[...]
        sh = b1_ref[...] - mean * sc
        gated = _gate_preact(x_ref[...], g_ref[...], nb_ref[...],
                             wa_ref[...], wb_ref[...], wc_ref[...], bf_ref[...])
        gn = gated * sc + sh
        nf = jax.nn.sigmoid(gn[:, :AFL])
        nc = _lrelu(gn[:, AFL:])
        p = (nf * nc).reshape(BA, M, AFL)
        s = jnp.sum(p, axis=1)
        s_ref[...] = s

        @pl.when(pl.program_id(0) == 0)
        def _():
            st2_ref[...] = jnp.zeros_like(st2_ref)

        st2_ref[...] += jnp.concatenate(
            [jnp.sum(s, axis=0, keepdims=True),
             jnp.sum(s * s, axis=0, keepdims=True)], axis=0)

    return pl.pallas_call(
        body,
        grid=(GRID,),
        in_specs=[
            pl.BlockSpec((BA, AFL), lambda i: (i, 0)),
            pl.BlockSpec((BE, AFL), lambda i: (i, 0)),
            pl.BlockSpec((NBL, BE), lambda i: (0, i)),
            pl.BlockSpec((AFL, 2 * AFL), lambda i: (0, 0)),
            pl.BlockSpec((AFL, 2 * AFL), lambda i: (0, 0)),
            pl.BlockSpec((NBL, 2 * AFL), lambda i: (0, 0)),
            pl.BlockSpec((1, 2 * AFL), lambda i: (0, 0)),
            pl.BlockSpec((2, 2 * AFL), lambda i: (0, 0)),
            pl.BlockSpec((1, 2 * AFL), lambda i: (0, 0)),
            pl.BlockSpec((1, 2 * AFL), lambda i: (0, 0)),
        ],
        out_specs=[
            pl.BlockSpec((BA, AFL), lambda i: (i, 0)),
            pl.BlockSpec((2, AFL), lambda i: (0, 0)),
        ],
        out_shape=[
            jax.ShapeDtypeStruct((N, AFL), jnp.float32),
            jax.ShapeDtypeStruct((2, AFL), jnp.float32),
        ],
        compiler_params=pltpu.CompilerParams(
            dimension_semantics=("arbitrary",)),
    )(x, g, nbrf2, WaT, WbT, WcT, bf, st1, g1, b1)


def _residual(x, s, st2, g2, b2):
    def body(x_ref, s_ref, st_ref, g2_ref, b2_ref, o_ref):
        mean = st_ref[0:1, :] / N
        var = st_ref[1:2, :] / N - mean * mean
        inv = lax.rsqrt(var + EPS)
        sn = (s_ref[...] - mean) * inv * g2_ref[...] + b2_ref[...]
        o_ref[...] = _lrelu(x_ref[...] + sn)

    return pl.pallas_call(
        body,
        grid=(GRID,),
        in_specs=[
            pl.BlockSpec((BA, AFL), lambda i: (i, 0)),
            pl.BlockSpec((BA, AFL), lambda i: (i, 0)),
            pl.BlockSpec((2, AFL), lambda i: (0, 0)),
            pl.BlockSpec((1, AFL), lambda i: (0, 0)),
            pl.BlockSpec((1, AFL), lambda i: (0, 0)),
        ],
        out_specs=pl.BlockSpec((BA, AFL), lambda i: (i, 0)),
        out_shape=jax.ShapeDtypeStruct((N, AFL), jnp.float32),
        compiler_params=pltpu.CompilerParams(
            dimension_semantics=("arbitrary",)),
    )(x, s, st2, g2, b2)


def _head(x, conn, WcT, bc, WvT, bv):
    def body(x_ref, c_ref, wc_ref, bc_ref, wv_ref, bv_ref, y_ref, v_ref):
        h = jnp.dot(x_ref[...], wc_ref[...],
                    preferred_element_type=jnp.float32) + bc_ref[...]
        y = jnp.dot(h, wv_ref[...],
                    preferred_element_type=jnp.float32) + bv_ref[...]
        y_ref[...] = y
        v_ref[...] = y * c_ref[...]

    return pl.pallas_call(
        body,
        grid=(GRID,),
        in_specs=[
            pl.BlockSpec((BA, AFL), lambda i: (i, 0)),
            pl.BlockSpec((BA, 1), lambda i: (i, 0)),
            pl.BlockSpec((AFL, 128), lambda i: (0, 0)),
            pl.BlockSpec((1, 128), lambda i: (0, 0)),
            pl.BlockSpec((128, 1), lambda i: (0, 0)),
            pl.BlockSpec((1, 1), lambda i: (0, 0)),
        ],
        out_specs=[
            pl.BlockSpec((BA, 1), lambda i: (i, 0)),
            pl.BlockSpec((BA, 1), lambda i: (i, 0)),
        ],
        out_shape=[
            jax.ShapeDtypeStruct((N, 1), jnp.float32),
            jax.ShapeDtypeStruct((N, 1), jnp.float32),
        ],
    )(x, conn, WcT, bc, WvT, bv)


def _pool(vis2):
    def body(v_ref, o_ref):
        v = v_ref[...]
        ssum = jnp.sum(v, axis=1, keepdims=True)
        cnt = jnp.sum((v != 0).astype(jnp.float32), axis=1, keepdims=True)
        o_ref[...] = ssum / cnt

    return pl.pallas_call(
        body,
        in_specs=[pl.BlockSpec((N0, A), lambda: (0, 0))],
        out_specs=pl.BlockSpec((N0, 1), lambda: (0, 0)),
        out_shape=jax.ShapeDtypeStruct((N0, 1), jnp.float32),
    )(vis2)


def kernel(atom_fea, nbr_fea, nbr_fea_idx, crystal_atom_idx, distances,
           connection_atom_idx, params):
    del distances, crystal_atom_idx

    flat_idx = nbr_fea_idx.reshape(-1)
    flat_idx = jnp.concatenate(
        [flat_idx, jnp.zeros((E_PAD - E,), jnp.int32)]).reshape(NW, KCH, CHUNK)
    nbrf2 = nbr_fea.reshape(E, NBL).T

    x = _embed(atom_fea, params['We'].T, params['be'][None, :])

    for i in range(3):
        Wf = params['Wf%d' % i]
        WaT = Wf[:, :AFL].T
        WbT = Wf[:, AFL:2 * AFL].T
        WcT = Wf[:, 2 * AFL:].T
        bf = params['bf%d' % i][None, :]
        g1 = params['g1_%d' % i][None, :]
        b1 = params['b1_%d' % i][None, :]
        g2 = params['g2_%d' % i][None, :]
        b2 = params['b2_%d' % i][None, :]

        g = _sc_gather(x, flat_idx)
        st1 = _stats1(x, g, nbrf2, WaT, WbT, WcT, bf)
        s, st2 = _conv_sum(x, g, nbrf2, WaT, WbT, WcT, bf, st1, g1, b1)
        x = _residual(x, s, st2, g2, b2)

    y, vis = _head(x, connection_atom_idx, params['Wc'].T,
                   params['bc'][None, :], params['Wv'].T,
                   params['bv'][None, None, 0])
    out = _pool(vis.reshape(N0, A))
    return out, vis, y


# speedup vs baseline: 1.2555x; 1.2555x over previous
import functools

import jax
import jax.numpy as jnp
import numpy as np
from jax import lax
from jax.experimental import pallas as pl
from jax.experimental.pallas import tpu as pltpu
from jax.experimental.pallas import tpu_sc as plsc

AFL = 32
NBL = 4
N = 50000
M = 16
N0 = 1000
A = 50
EPS = 1e-5

NW = 32
CHUNK = 128
KCH = 196
PER_W = KCH * CHUNK
E_PAD = NW * PER_W
E = N * M

BA = 1024
BE = BA * M
BR = BE // 4
NPAD = NW * PER_W // M
RP = E // 4
RPAD = E_PAD // 4
GRID = NPAD // BA
AVALID = N - (GRID - 1) * BA
RVALID = RP - (GRID - 1) * BR


_PERM = np.array([64 * q + f for q in range(4) for f in range(32)]
                 + [64 * q + 32 + f for q in range(4) for f in range(32)])


def _lrelu(x):
    return jnp.maximum(x, 0.01 * x)


def _bdiag4(w):
    a, b = w.shape
    out = jnp.zeros((4 * a, 4 * b), w.dtype)
    for q in range(4):
        out = out.at[q * a:(q + 1) * a, q * b:(q + 1) * b].set(w)
    return out


def _sc_gather(x, idx3d):
    mesh = plsc.VectorSubcoreMesh(core_axis_name="c", subcore_axis_name="s")

    @functools.partial(
        pl.kernel,
        mesh=mesh,
        out_type=jax.ShapeDtypeStruct((E_PAD, AFL), jnp.float32),
        scratch_types=[
            pltpu.VMEM((KCH, CHUNK), jnp.int32),
            pltpu.VMEM((CHUNK, AFL), jnp.float32),
            pltpu.VMEM((CHUNK, AFL), jnp.float32),
            pltpu.SemaphoreType.DMA,
            pltpu.SemaphoreType.DMA,
        ],
        compiler_params=pltpu.CompilerParams(use_tc_tiling_on_sc=False),
    )
    def k(x_hbm, idx_hbm, g_hbm, idx_v, rows0, rows1, sem0, sem1):
        wid = lax.axis_index("s") * 2 + lax.axis_index("c")
        pltpu.sync_copy(idx_hbm.at[wid], idx_v)
        base = wid * PER_W

        def issue(j, rows, sem):
            return pltpu.async_copy(x_hbm.at[idx_v.at[j]], rows, sem)

        def drain(j, rows):
            pltpu.sync_copy(rows, g_hbm.at[pl.ds(base + j * CHUNK, CHUNK)])

        c0 = issue(0, rows0, sem0)

        def body(jj, carry):
            j = jj * 2

            c_next = issue(j + 1, rows1, sem1)
            c0 = pltpu.make_async_copy(x_hbm.at[idx_v.at[j]], rows0, sem0)
            c0.wait()
            drain(j, rows0)

            c_next2 = issue(j + 2, rows0, sem0)
            c1 = pltpu.make_async_copy(x_hbm.at[idx_v.at[j + 1]], rows1, sem1)
            c1.wait()
            drain(j + 1, rows1)
            return carry

        lax.fori_loop(0, (KCH - 2) // 2, body, 0, unroll=False)
        j = KCH - 2
        c_last = issue(j + 1, rows1, sem1)
        pltpu.make_async_copy(x_hbm.at[idx_v.at[j]], rows0, sem0).wait()
        drain(j, rows0)
        pltpu.make_async_copy(x_hbm.at[idx_v.at[j + 1]], rows1, sem1).wait()
        drain(j + 1, rows1)

    return k(x, idx3d)


def _embed(atom_fea, WeT, be):
    def body(a_ref, w_ref, b_ref, o_ref):
        o_ref[...] = jnp.dot(a_ref[...], w_ref[...],
                             preferred_element_type=jnp.float32) + b_ref[...]

    return pl.pallas_call(
        body,
        grid=(GRID,),
        in_specs=[
            pl.BlockSpec((BA, 128), lambda i: (i, 0)),
            pl.BlockSpec((128, AFL), lambda i: (0, 0)),
            pl.BlockSpec((1, AFL), lambda i: (0, 0)),
        ],
        out_specs=pl.BlockSpec((BA, AFL), lambda i: (i, 0)),
        out_shape=jax.ShapeDtypeStruct((NPAD, AFL), jnp.float32),
    )(atom_fea, WeT, be)


def _gate_packed(x_blk, gp_blk, nbp_blk, wa4, wb4, wc4, bf4):
    self_g = jnp.dot(x_blk, wa4, preferred_element_type=jnp.float32)
    sgt = jnp.broadcast_to(self_g[:, None, :], (BA, 4, 8 * AFL))
    sgt = sgt.reshape(BR, 8 * AFL)
    nb_term = lax.dot_general(nbp_blk, wc4, (((0,), (0,)), ((), ())),
                              preferred_element_type=jnp.float32)
    edge = (jnp.dot(gp_blk, wb4, preferred_element_type=jnp.float32)
            + nb_term + bf4)
    return sgt + edge


def _fold4(v):
    k = v.shape[1] // 4
    return v[:, 0:k] + v[:, k:2 * k] + v[:, 2 * k:3 * k] + v[:, 3 * k:4 * k]


def _foldstat(v):
    return jnp.concatenate(
        [_fold4(v[:, :4 * AFL]), _fold4(v[:, 4 * AFL:])], axis=1)


def _stats1(x, gp, nbp, WaT, wb4, wc4, bf4):
    def body(x_ref, g_ref, nb_ref, wa_ref, wb_ref, wc_ref, bf_ref, st_ref):
        gated = _gate_packed(x_ref[...], g_ref[...], nb_ref[...],
                             wa_ref[...], wb_ref[...], wc_ref[...], bf_ref[...])

        @pl.when(pl.program_id(0) == 0)
        def _():
            st_ref[...] = jnp.zeros_like(st_ref)

        @pl.when(pl.program_id(0) == GRID - 1)
        def _():
            rows = lax.broadcasted_iota(jnp.int32, (BR, 1), 0)
            gm = jnp.where(rows < RVALID, gated, 0.0)
            s1 = _foldstat(jnp.sum(gm, axis=0, keepdims=True))
            s2 = _foldstat(jnp.sum(gm * gm, axis=0, keepdims=True))
            st_ref[...] += jnp.concatenate([s1, s2], axis=0)

        @pl.when(pl.program_id(0) < GRID - 1)
        def _():
            s1 = _foldstat(jnp.sum(gated, axis=0, keepdims=True))
            s2 = _foldstat(jnp.sum(gated * gated, axis=0, keepdims=True))
            st_ref[...] += jnp.concatenate([s1, s2], axis=0)

    return pl.pallas_call(
        body,
        grid=(GRID,),
        in_specs=[
            pl.BlockSpec((BA, AFL), lambda i: (i, 0)),
            pl.BlockSpec((BR, 128), lambda i: (i, 0)),
            pl.BlockSpec((4 * NBL, BR), lambda i: (0, i)),
            pl.BlockSpec((AFL, 8 * AFL), lambda i: (0, 0)),
            pl.BlockSpec((128, 8 * AFL), lambda i: (0, 0)),
            pl.BlockSpec((4 * NBL, 8 * AFL), lambda i: (0, 0)),
            pl.BlockSpec((1, 8 * AFL), lambda i: (0, 0)),
        ],
        out_specs=pl.BlockSpec((2, 2 * AFL), lambda i: (0, 0)),
        out_shape=jax.ShapeDtypeStruct((2, 2 * AFL), jnp.float32),
        compiler_params=pltpu.CompilerParams(
            dimension_semantics=("arbitrary",)),
    )(x, gp, nbp, WaT, wb4, wc4, bf4)


def _conv_sum(x, gp, nbp, WaT, wb4, wc4, bf4, st1, g1, b1):
    def body(x_ref, g_ref, nb_ref, wa_ref, wb_ref, wc_ref, bf_ref,
             st_ref, g1_ref, b1_ref, s_ref, st2_ref):
        mean = st_ref[0:1, :] / E
        var = st_ref[1:2, :] / E - mean * mean
        sc = lax.rsqrt(var + EPS) * g1_ref[...]
        sh = b1_ref[...] - mean * sc
        sc4 = jnp.concatenate([sc[:, :AFL]] * 4 + [sc[:, AFL:]] * 4, axis=1)
        sh4 = jnp.concatenate([sh[:, :AFL]] * 4 + [sh[:, AFL:]] * 4, axis=1)
        gated = _gate_packed(x_ref[...], g_ref[...], nb_ref[...],
                             wa_ref[...], wb_ref[...], wc_ref[...], bf_ref[...])
        gn = gated * sc4 + sh4
        nf = jax.nn.sigmoid(gn[:, :4 * AFL])
        nc = _lrelu(gn[:, 4 * AFL:])
        pp = nf * nc
        r = jnp.sum(pp.reshape(BA, 4, 128), axis=1)
        s = (r[:, 0:AFL] + r[:, AFL:2 * AFL]
             + r[:, 2 * AFL:3 * AFL] + r[:, 3 * AFL:4 * AFL])
        s_ref[...] = s

        @pl.when(pl.program_id(0) == 0)
        def _():
            st2_ref[...] = jnp.zeros_like(st2_ref)

        @pl.when(pl.program_id(0) == GRID - 1)
        def _():
            rows = lax.broadcasted_iota(jnp.int32, (BA, 1), 0)
            sm = jnp.where(rows < AVALID, s, 0.0)
            st2_ref[...] += jnp.concatenate(
                [jnp.sum(sm, axis=0, keepdims=True),
                 jnp.sum(sm * sm, axis=0, keepdims=True)], axis=0)

        @pl.when(pl.program_id(0) < GRID - 1)
        def _():
            st2_ref[...] += jnp.concatenate(
                [jnp.sum(s, axis=0, keepdims=True),
                 jnp.sum(s * s, axis=0, keepdims=True)], axis=0)

    return pl.pallas_call(
        body,
        grid=(GRID,),
        in_specs=[
            pl.BlockSpec((BA, AFL), lambda i: (i, 0)),
            pl.BlockSpec((BR, 128), lambda i: (i, 0)),
            pl.BlockSpec((4 * NBL, BR), lambda i: (0, i)),
            pl.BlockSpec((AFL, 8 * AFL), lambda i: (0, 0)),
            pl.BlockSpec((128, 8 * AFL), lambda i: (0, 0)),
            pl.BlockSpec((4 * NBL, 8 * AFL), lambda i: (0, 0)),
            pl.BlockSpec((1, 8 * AFL), lambda i: (0, 0)),
            pl.BlockSpec((2, 2 * AFL), lambda i: (0, 0)),
            pl.BlockSpec((1, 2 * AFL), lambda i: (0, 0)),
            pl.BlockSpec((1, 2 * AFL), lambda i: (0, 0)),
        ],
        out_specs=[
            pl.BlockSpec((BA, AFL), lambda i: (i, 0)),
            pl.BlockSpec((2, AFL), lambda i: (0, 0)),
        ],
        out_shape=[
            jax.ShapeDtypeStruct((NPAD, AFL), jnp.float32),
            jax.ShapeDtypeStruct((2, AFL), jnp.float32),
        ],
        compiler_params=pltpu.CompilerParams(
            dimension_semantics=("arbitrary",)),
    )(x, gp, nbp, WaT, wb4, wc4, bf4, st1, g1, b1)


def _residual(x, s, st2, g2, b2):
    def body(x_ref, s_ref, st_ref, g2_ref, b2_ref, o_ref):
        mean = st_ref[0:1, :] / N
        var = st_ref[1:2, :] / N - mean * mean
        inv = lax.rsqrt(var + EPS)
        sn = (s_ref[...] - mean) * inv * g2_ref[...] + b2_ref[...]
        o_ref[...] = _lrelu(x_ref[...] + sn)

    return pl.pallas_call(
        body,
        grid=(GRID,),
        in_specs=[
            pl.BlockSpec((BA, AFL), lambda i: (i, 0)),
            pl.BlockSpec((BA, AFL), lambda i: (i, 0)),
            pl.BlockSpec((2, AFL), lambda i: (0, 0)),
            pl.BlockSpec((1, AFL), lambda i: (0, 0)),
            pl.BlockSpec((1, AFL), lambda i: (0, 0)),
        ],
        out_specs=pl.BlockSpec((BA, AFL), lambda i: (i, 0)),
        out_shape=jax.ShapeDtypeStruct((NPAD, AFL), jnp.float32),
        compiler_params=pltpu.CompilerParams(
            dimension_semantics=("arbitrary",)),
    )(x, s, st2, g2, b2)


def _head(x, conn, WcT, bc, WvT, bv):
    def body(x_ref, c_ref, wc_ref, bc_ref, wv_ref, bv_ref, y_ref, v_ref):
        h = jnp.dot(x_ref[...], wc_ref[...],
                    preferred_element_type=jnp.float32) + bc_ref[...]
        y = jnp.dot(h, wv_ref[...],
                    preferred_element_type=jnp.float32) + bv_ref[...]
        y_ref[...] = y
        v_ref[...] = y * c_ref[...]

    return pl.pallas_call(
        body,
        grid=(GRID,),
        in_specs=[
            pl.BlockSpec((BA, AFL), lambda i: (i, 0)),
            pl.BlockSpec((BA, 1), lambda i: (i, 0)),
            pl.BlockSpec((AFL, 128), lambda i: (0, 0)),
            pl.BlockSpec((1, 128), lambda i: (0, 0)),
            pl.BlockSpec((128, 1), lambda i: (0, 0)),
            pl.BlockSpec((1, 1), lambda i: (0, 0)),
        ],
        out_specs=[
            pl.BlockSpec((BA, 1), lambda i: (i, 0)),
            pl.BlockSpec((BA, 1), lambda i: (i, 0)),
        ],
        out_shape=[
            jax.ShapeDtypeStruct((NPAD, 1), jnp.float32),
            jax.ShapeDtypeStruct((NPAD, 1), jnp.float32),
        ],
    )(x, conn, WcT, bc, WvT, bv)


def _pool(vis2):
    def body(v_ref, o_ref):
        v = v_ref[...]
        ssum = jnp.sum(v, axis=1, keepdims=True)
        cnt = jnp.sum((v != 0).astype(jnp.float32), axis=1, keepdims=True)
        o_ref[...] = ssum / cnt

    return pl.pallas_call(
        body,
        in_specs=[pl.BlockSpec((N0, A), lambda: (0, 0))],
        out_specs=pl.BlockSpec((N0, 1), lambda: (0, 0)),
        out_shape=jax.ShapeDtypeStruct((N0, 1), jnp.float32),
    )(vis2)


def kernel(atom_fea, nbr_fea, nbr_fea_idx, crystal_atom_idx, distances,
           connection_atom_idx, params):
    del distances, crystal_atom_idx

    flat_idx = nbr_fea_idx.reshape(-1)
    flat_idx = jnp.concatenate(
        [flat_idx, jnp.zeros((E_PAD - E,), jnp.int32)]).reshape(NW, KCH, CHUNK)
    nbp = nbr_fea.reshape(RP, 4, NBL).transpose(1, 2, 0).reshape(4 * NBL, RP)

    af_pad = jnp.concatenate(
        [atom_fea, jnp.zeros((NPAD - N, atom_fea.shape[1]), jnp.float32)])
    conn_pad = jnp.concatenate(
        [connection_atom_idx, jnp.zeros((NPAD - N, 1), jnp.float32)])

    x = _embed(af_pad, params['We'].T, params['be'][None, :])

    for i in range(3):
        Wf = params['Wf%d' % i]
        WaT = Wf[:, :AFL].T
        wa4 = jnp.concatenate([WaT[:, :AFL]] * 4 + [WaT[:, AFL:]] * 4, axis=1)
        wb4 = _bdiag4(Wf[:, AFL:2 * AFL].T)[:, _PERM]
        wc4 = _bdiag4(Wf[:, 2 * AFL:].T)[:, _PERM]
        bf4 = jnp.tile(params['bf%d' % i][None, :], (1, 4))[:, _PERM]
        g1 = params['g1_%d' % i][None, :]
        b1 = params['b1_%d' % i][None, :]
        g2 = params['g2_%d' % i][None, :]
        b2 = params['b2_%d' % i][None, :]

        g = _sc_gather(x, flat_idx)
        gp = g.reshape(RPAD, 128)
        st1 = _stats1(x, gp, nbp, wa4, wb4, wc4, bf4)
        s, st2 = _conv_sum(x, gp, nbp, wa4, wb4, wc4, bf4, st1, g1, b1)
        x = _residual(x, s, st2, g2, b2)

    y, vis = _head(x, conn_pad, params['Wc'].T,
                   params['bc'][None, :], params['Wv'].T,
                   params['bv'][None, None, 0])
    y = y[:N]
    vis = vis[:N]
    out = _pool(vis.reshape(N0, A))
    return out, vis, y
